# baseline (speedup 1.0000x reference)
import functools

import jax
import jax.numpy as jnp
from jax import lax
from jax.experimental import pallas as pl
from jax.experimental.pallas import tpu as pltpu
from jax.experimental.pallas import tpu_sc as plsc

N = 10000
E = 320000
D_IN = 128
D_HID = 16
D_OUT = 64

N_PAD = 10240
CHUNK = 128
CH_FULL = 78
TAIL = 16
EROWS = 2500

_mesh = plsc.VectorSubcoreMesh(core_axis_name="c", subcore_axis_name="s")
_sc_params = pltpu.CompilerParams(use_tc_tiling_on_sc=False)


_GRP = 6
_NG = CH_FULL // _GRP
_W = _GRP * CHUNK


def _stage_indices(ei_hbm, eit_hbm, row, w, idx_v, tail_v, sem):
    pltpu.async_copy(ei_hbm.at[row, pl.ds(w * _NG, _NG)], idx_v, sem)
    pltpu.async_copy(eit_hbm.at[row, pl.ds(w * TAIL, TAIL)], tail_v, sem)


def _wait_indices(ei_hbm, eit_hbm, row, w, idx_v, tail_v, sem):
    pltpu.make_async_copy(ei_hbm.at[row, pl.ds(w * _NG, _NG)], idx_v,
                          sem).wait()
    pltpu.make_async_copy(eit_hbm.at[row, pl.ds(w * TAIL, TAIL)], tail_v,
                          sem).wait()


@functools.partial(
    pl.kernel,
    out_type=jax.ShapeDtypeStruct((2, N_PAD), jnp.float32),
    mesh=_mesh,
    scratch_types=[
        pltpu.VMEM((_NG, _W), jnp.int32),
        pltpu.VMEM((TAIL,), jnp.int32),
        pltpu.VMEM((_W,), jnp.float32),
        pltpu.VMEM((TAIL,), jnp.float32),
        pltpu.VMEM((640,), jnp.float32),
        pltpu.VMEM_SHARED((N_PAD,), jnp.float32),
        pltpu.SemaphoreType.DMA,
        pltpu.SemaphoreType.DMA,
    ],
    compiler_params=_sc_params,
)
def _count_kernel(ei_hbm, eit_hbm, out_hbm, didx_v, dtail_v, ones_v, ones_t,
                  zbuf_v, acc_sh, isem, ssem):
    c = lax.axis_index("c")
    s = lax.axis_index("s")
    w = c * 16 + s
    _stage_indices(ei_hbm, eit_hbm, 1, w, didx_v, dtail_v, isem)

    def fill_body(i, _):
        ones_v[pl.ds(i * 16, 16)] = jnp.ones((16,), jnp.float32)
        return 0
    lax.fori_loop(0, _W // 16, fill_body, 0)
    ones_t[...] = jnp.ones((16,), jnp.float32)

    def zero_body(i, _):
        zbuf_v[pl.ds(i * 16, 16)] = jnp.zeros((16,), jnp.float32)
        return 0
    lax.fori_loop(0, 640 // 16, zero_body, 0)

    _wait_indices(ei_hbm, eit_hbm, 1, w, didx_v, dtail_v, isem)
    pltpu.sync_copy(zbuf_v, acc_sh.at[pl.ds(s * 640, 640)])
    plsc.subcore_barrier()

    def cnt_body(j, _):
        pltpu.make_async_copy(ones_v, acc_sh.at[didx_v.at[j]], ssem).wait()
        pltpu.async_copy(ones_v, acc_sh.at[didx_v.at[j + 2]], ssem, add=True)
        return 0

    for j0 in range(2):
        pltpu.async_copy(ones_v, acc_sh.at[didx_v.at[j0]], ssem, add=True)
    lax.fori_loop(0, _NG - 2, cnt_body, 0)

    def drain_body(j, _):
        pltpu.make_async_copy(ones_v, acc_sh.at[didx_v.at[j]], ssem).wait()
        return 0
    lax.fori_loop(0, 2, drain_body, 0)
    pltpu.sync_copy(ones_t, acc_sh.at[dtail_v], add=True)
    plsc.subcore_barrier()

    pltpu.sync_copy(acc_sh.at[pl.ds(s * 640, 640)],
                    out_hbm.at[c].at[pl.ds(s * 640, 640)])


def _make_agg_kernel(d_feat, subw, nbuf):
    spr = _W // subw
    ng = _NG * spr

    @functools.partial(
        pl.kernel,
        out_type=jax.ShapeDtypeStruct((2, N_PAD, d_feat), jnp.float32),
        mesh=_mesh,
        scratch_types=[
            pltpu.VMEM((_NG, _W), jnp.int32),
            pltpu.VMEM((_NG, _W), jnp.int32),
            pltpu.VMEM((TAIL,), jnp.int32),
            pltpu.VMEM((TAIL,), jnp.int32),
            pltpu.VMEM((nbuf, subw, d_feat), jnp.float32),
            pltpu.VMEM((TAIL, d_feat), jnp.float32),
            pltpu.VMEM((64, d_feat), jnp.float32),
            pltpu.VMEM_SHARED((N_PAD, d_feat), jnp.float32),
            pltpu.SemaphoreType.DMA,
        ] + [pltpu.SemaphoreType.DMA] * nbuf,
        compiler_params=_sc_params,
    )
    def agg(ei_hbm, eit_hbm, g_hbm, out_hbm, sidx_v, didx_v, stail_v, dtail_v,
            rows_v, trows_v, zbuf_v, acc_sh, isem, *gsem):
        c = lax.axis_index("c")
        s = lax.axis_index("s")
        w = c * 16 + s
        _stage_indices(ei_hbm, eit_hbm, 0, w, sidx_v, stail_v, isem)
        _stage_indices(ei_hbm, eit_hbm, 1, w, didx_v, dtail_v, isem)

        def zzero(i, _):
            j = i // (d_feat // 16)
            k = i % (d_feat // 16)
            zbuf_v[j, pl.ds(k * 16, 16)] = jnp.zeros((16,), jnp.float32)
            return 0
        lax.fori_loop(0, 64 * (d_feat // 16), zzero, 0)

        _wait_indices(ei_hbm, eit_hbm, 0, w, sidx_v, stail_v, isem)
        _wait_indices(ei_hbm, eit_hbm, 1, w, didx_v, dtail_v, isem)

        for t in range(10):
            pltpu.async_copy(zbuf_v, acc_sh.at[pl.ds(s * 640 + t * 64, 64)],
                             isem)
        for t in range(10):
            pltpu.make_async_copy(
                zbuf_v, acc_sh.at[pl.ds(s * 640 + t * 64, 64)], isem).wait()
        plsc.subcore_barrier()

        def sl(idx_v, g):
            return idx_v.at[g // spr, pl.ds((g % spr) * subw, subw)]

        def issue(g, b):
            pltpu.async_copy(g_hbm.at[sl(sidx_v, g)], rows_v.at[b], gsem[b])

        def drain(g, b):
            pltpu.make_async_copy(g_hbm.at[sl(sidx_v, g)], rows_v.at[b],
                                  gsem[b]).wait()
            pltpu.sync_copy(rows_v.at[b], acc_sh.at[sl(didx_v, g)], add=True)

        for b in range(nbuf):
            issue(b, b)

        k_full = ng // nbuf - 1

        def outer_body(o, _):
            for b in range(nbuf):
                i = o * nbuf + b
                drain(i, b)
                issue(i + nbuf, b)
            return 0
        lax.fori_loop(0, k_full, outer_body, 0)

        for i in range(k_full * nbuf, ng):
            b = i % nbuf
            drain(i, b)
            if i + nbuf < ng:
                issue(i + nbuf, b)

        pltpu.async_copy(g_hbm.at[stail_v], trows_v, isem)
        pltpu.make_async_copy(g_hbm.at[stail_v], trows_v, isem).wait()
        pltpu.sync_copy(trows_v, acc_sh.at[dtail_v], add=True)
        plsc.subcore_barrier()

        pltpu.sync_copy(acc_sh.at[pl.ds(s * 640, 640)],
                        out_hbm.at[c].at[pl.ds(s * 640, 640)])
    return agg


_agg16 = _make_agg_kernel(D_HID, _W, 2)
_agg64 = _make_agg_kernel(D_OUT, _W // 2, 2)


def _h1_body(x_ref, w1_ref, out_ref):
    out_ref[...] = jnp.dot(x_ref[...], w1_ref[...],
                           preferred_element_type=jnp.float32)


_h1_kernel = pl.pallas_call(
    _h1_body,
    out_shape=jax.ShapeDtypeStruct((N, D_HID), jnp.float32),
)


def _scale_body(cnt_ref, h1_ref, g1_ref, dis_ref):
    deg = jnp.sum(cnt_ref[...], axis=0, keepdims=True) + 1.0
    dis_col = jnp.transpose(lax.rsqrt(deg), (1, 0))
    dis_ref[...] = dis_col
    g1_ref[...] = dis_col[:N] * h1_ref[...]


_scale_kernel = pl.pallas_call(
    _scale_body,
    out_shape=(jax.ShapeDtypeStruct((N, D_HID), jnp.float32),
               jax.ShapeDtypeStruct((N_PAD, 1), jnp.float32)),
)


def _mid_body(p_ref, g1_ref, dis_ref, b1_ref, w2_ref, out_ref):
    agg = p_ref[0, :N, :] + p_ref[1, :N, :]
    a1 = jnp.maximum(dis_ref[...] * (agg + g1_ref[...]) + b1_ref[...], 0.0)
    h2 = jnp.dot(a1, w2_ref[...], preferred_element_type=jnp.float32)
    out_ref[...] = dis_ref[...] * h2


_mid_kernel = pl.pallas_call(
    _mid_body,
    out_shape=jax.ShapeDtypeStruct((N, D_OUT), jnp.float32),
)


def _fin_body(q_ref, g2_ref, dis_ref, b2_ref, out_ref):
    agg = q_ref[0, :N, :] + q_ref[1, :N, :]
    out_ref[...] = dis_ref[...] * (agg + g2_ref[...]) + b2_ref[...]


_fin_kernel = pl.pallas_call(
    _fin_body,
    out_shape=jax.ShapeDtypeStruct((N, D_OUT), jnp.float32),
)


def kernel(x, edge_index, W1, b1, W2, b2):
    ei32 = edge_index.astype(jnp.int32)
    ei = ei32[:, :32 * _NG * _W].reshape(2, 32 * _NG, _W)
    eit = ei32[:, 32 * _NG * _W:]

    cntp = _count_kernel(ei, eit)
    h1 = _h1_kernel(x, W1)
    g1, dis_col_p = _scale_kernel(cntp, h1)
    dis_col = dis_col_p[:N]

    p1 = _agg16(ei, eit, g1)
    g2 = _mid_kernel(p1, g1, dis_col, b1.reshape(1, D_HID), W2)
    p2 = _agg64(ei, eit, g2)
    out = _fin_kernel(p2, g2, dis_col, b2.reshape(1, D_OUT))
    return out


# speedup vs baseline: 1.0886x; 1.0886x over previous
import functools

import jax
import jax.numpy as jnp
from jax import lax
from jax.experimental import pallas as pl
from jax.experimental.pallas import tpu as pltpu
from jax.experimental.pallas import tpu_sc as plsc

N = 10000
E = 320000
D_IN = 128
D_HID = 16
D_OUT = 64

N_PAD = 10240
CHUNK = 128
CH_FULL = 78
TAIL = 16
EROWS = 2500

_mesh = plsc.VectorSubcoreMesh(core_axis_name="c", subcore_axis_name="s")
_sc_params = pltpu.CompilerParams(use_tc_tiling_on_sc=False)


_GRP = 6
_NG = CH_FULL // _GRP
_W = _GRP * CHUNK


def _stage_indices(ei_hbm, eit_hbm, row, w, idx_v, tail_v, sem):
    pltpu.async_copy(ei_hbm.at[row, pl.ds(w * _NG, _NG)], idx_v, sem)
    pltpu.async_copy(eit_hbm.at[row, pl.ds(w * TAIL, TAIL)], tail_v, sem)


def _wait_indices(ei_hbm, eit_hbm, row, w, idx_v, tail_v, sem):
    pltpu.make_async_copy(ei_hbm.at[row, pl.ds(w * _NG, _NG)], idx_v,
                          sem).wait()
    pltpu.make_async_copy(eit_hbm.at[row, pl.ds(w * TAIL, TAIL)], tail_v,
                          sem).wait()


@functools.partial(
    pl.kernel,
    out_type=jax.ShapeDtypeStruct((2, N_PAD), jnp.float32),
    mesh=_mesh,
    scratch_types=[
        pltpu.VMEM((_NG, _W), jnp.int32),
        pltpu.VMEM((TAIL,), jnp.int32),
        pltpu.VMEM((_W,), jnp.float32),
        pltpu.VMEM((TAIL,), jnp.float32),
        pltpu.VMEM((640,), jnp.float32),
        pltpu.VMEM_SHARED((N_PAD,), jnp.float32),
        pltpu.SemaphoreType.DMA,
        pltpu.SemaphoreType.DMA,
    ],
    compiler_params=_sc_params,
)
def _count_kernel(ei_hbm, eit_hbm, out_hbm, didx_v, dtail_v, ones_v, ones_t,
                  zbuf_v, acc_sh, isem, ssem):
    c = lax.axis_index("c")
    s = lax.axis_index("s")
    w = c * 16 + s
    _stage_indices(ei_hbm, eit_hbm, 1, w, didx_v, dtail_v, isem)

    def fill_body(i, _):
        ones_v[pl.ds(i * 16, 16)] = jnp.ones((16,), jnp.float32)
        return 0
    lax.fori_loop(0, _W // 16, fill_body, 0)
    ones_t[...] = jnp.ones((16,), jnp.float32)

    def zero_body(i, _):
        zbuf_v[pl.ds(i * 16, 16)] = jnp.zeros((16,), jnp.float32)
        return 0
    lax.fori_loop(0, 640 // 16, zero_body, 0)

    _wait_indices(ei_hbm, eit_hbm, 1, w, didx_v, dtail_v, isem)
    pltpu.sync_copy(zbuf_v, acc_sh.at[pl.ds(s * 640, 640)])
    plsc.subcore_barrier()

    def cnt_body(j, _):
        pltpu.make_async_copy(ones_v, acc_sh.at[didx_v.at[j]], ssem).wait()
        pltpu.async_copy(ones_v, acc_sh.at[didx_v.at[j + 2]], ssem, add=True)
        return 0

    for j0 in range(2):
        pltpu.async_copy(ones_v, acc_sh.at[didx_v.at[j0]], ssem, add=True)
    lax.fori_loop(0, _NG - 2, cnt_body, 0)

    def drain_body(j, _):
        pltpu.make_async_copy(ones_v, acc_sh.at[didx_v.at[j]], ssem).wait()
        return 0
    lax.fori_loop(0, 2, drain_body, 0)
    pltpu.sync_copy(ones_t, acc_sh.at[dtail_v], add=True)
    plsc.subcore_barrier()

    pltpu.sync_copy(acc_sh.at[pl.ds(s * 640, 640)],
                    out_hbm.at[c].at[pl.ds(s * 640, 640)])


def _make_agg_kernel(d_feat, subw, nbuf):
    spr = _W // subw
    ng = _NG * spr

    @functools.partial(
        pl.kernel,
        out_type=jax.ShapeDtypeStruct((2, N_PAD, d_feat), jnp.float32),
        mesh=_mesh,
        scratch_types=[
            pltpu.VMEM((_NG, _W), jnp.int32),
            pltpu.VMEM((_NG, _W), jnp.int32),
            pltpu.VMEM((TAIL,), jnp.int32),
            pltpu.VMEM((TAIL,), jnp.int32),
            pltpu.VMEM((nbuf, subw, d_feat), jnp.float32),
            pltpu.VMEM((TAIL, d_feat), jnp.float32),
            pltpu.VMEM((64, d_feat), jnp.float32),
            pltpu.VMEM_SHARED((N_PAD, d_feat), jnp.float32),
            pltpu.SemaphoreType.DMA,
        ] + [pltpu.SemaphoreType.DMA] * nbuf,
        compiler_params=_sc_params,
    )
    def agg(ei_hbm, eit_hbm, g_hbm, out_hbm, sidx_v, didx_v, stail_v, dtail_v,
            rows_v, trows_v, zbuf_v, acc_sh, isem, *gsem):
        c = lax.axis_index("c")
        s = lax.axis_index("s")
        w = c * 16 + s
        _stage_indices(ei_hbm, eit_hbm, 0, w, sidx_v, stail_v, isem)
        _stage_indices(ei_hbm, eit_hbm, 1, w, didx_v, dtail_v, isem)

        def zzero(i, _):
            j = i // (d_feat // 16)
            k = i % (d_feat // 16)
            zbuf_v[j, pl.ds(k * 16, 16)] = jnp.zeros((16,), jnp.float32)
            return 0
        lax.fori_loop(0, 64 * (d_feat // 16), zzero, 0)

        _wait_indices(ei_hbm, eit_hbm, 0, w, sidx_v, stail_v, isem)
        _wait_indices(ei_hbm, eit_hbm, 1, w, didx_v, dtail_v, isem)

        for t in range(10):
            pltpu.async_copy(zbuf_v, acc_sh.at[pl.ds(s * 640 + t * 64, 64)],
                             isem)
        for t in range(10):
            pltpu.make_async_copy(
                zbuf_v, acc_sh.at[pl.ds(s * 640 + t * 64, 64)], isem).wait()
        plsc.subcore_barrier()

        def sl(idx_v, g):
            return idx_v.at[g // spr, pl.ds((g % spr) * subw, subw)]

        def issue(g, b):
            pltpu.async_copy(g_hbm.at[sl(sidx_v, g)], rows_v.at[b], gsem[b])

        def drain(g, b):
            pltpu.make_async_copy(g_hbm.at[sl(sidx_v, g)], rows_v.at[b],
                                  gsem[b]).wait()
            pltpu.sync_copy(rows_v.at[b], acc_sh.at[sl(didx_v, g)], add=True)

        for b in range(nbuf):
            issue(b, b)

        k_full = ng // nbuf - 1

        def outer_body(o, _):
            for b in range(nbuf):
                i = o * nbuf + b
                drain(i, b)
                issue(i + nbuf, b)
            return 0
        lax.fori_loop(0, k_full, outer_body, 0)

        for i in range(k_full * nbuf, ng):
            b = i % nbuf
            drain(i, b)
            if i + nbuf < ng:
                issue(i + nbuf, b)

        pltpu.async_copy(g_hbm.at[stail_v], trows_v, isem)
        pltpu.make_async_copy(g_hbm.at[stail_v], trows_v, isem).wait()
        pltpu.sync_copy(trows_v, acc_sh.at[dtail_v], add=True)
        plsc.subcore_barrier()

        pltpu.sync_copy(acc_sh.at[pl.ds(s * 640, 640)],
                        out_hbm.at[c].at[pl.ds(s * 640, 640)])
    return agg


_agg16 = _make_agg_kernel(D_HID, _W, 2)
_agg64 = _make_agg_kernel(D_OUT, _W // 6, 6)


def _h1_body(x_ref, w1_ref, out_ref):
    out_ref[...] = jnp.dot(x_ref[...], w1_ref[...],
                           preferred_element_type=jnp.float32)


_h1_kernel = pl.pallas_call(
    _h1_body,
    out_shape=jax.ShapeDtypeStruct((N, D_HID), jnp.float32),
)


def _scale_body(cnt_ref, h1_ref, g1_ref, dis_ref):
    deg = jnp.sum(cnt_ref[...], axis=0, keepdims=True) + 1.0
    dis_col = jnp.transpose(lax.rsqrt(deg), (1, 0))
    dis_ref[...] = dis_col
    g1_ref[...] = dis_col[:N] * h1_ref[...]


_scale_kernel = pl.pallas_call(
    _scale_body,
    out_shape=(jax.ShapeDtypeStruct((N, D_HID), jnp.float32),
               jax.ShapeDtypeStruct((N_PAD, 1), jnp.float32)),
)


def _mid_body(p_ref, g1_ref, dis_ref, b1_ref, w2_ref, out_ref):
    agg = p_ref[0, :N, :] + p_ref[1, :N, :]
    a1 = jnp.maximum(dis_ref[...] * (agg + g1_ref[...]) + b1_ref[...], 0.0)
    h2 = jnp.dot(a1, w2_ref[...], preferred_element_type=jnp.float32)
    out_ref[...] = dis_ref[...] * h2


_mid_kernel = pl.pallas_call(
    _mid_body,
    out_shape=jax.ShapeDtypeStruct((N, D_OUT), jnp.float32),
)


_QROWS = N_PAD * D_OUT // 128
_FROWS = N * D_OUT // 128


def _fin_body(q_ref, g2_ref, dis_ref, b2_ref, out_ref):
    agg = q_ref[0, :_FROWS, :] + q_ref[1, :_FROWS, :]
    out_ref[...] = dis_ref[...] * (agg + g2_ref[...]) + b2_ref[...]


_fin_kernel = pl.pallas_call(
    _fin_body,
    out_shape=jax.ShapeDtypeStruct((_FROWS, 128), jnp.float32),
)


def kernel(x, edge_index, W1, b1, W2, b2):
    ei32 = edge_index.astype(jnp.int32)
    ei = ei32[:, :32 * _NG * _W].reshape(2, 32 * _NG, _W)
    eit = ei32[:, 32 * _NG * _W:]

    cntp = _count_kernel(ei, eit)
    h1 = _h1_kernel(x, W1)
    g1, dis_col_p = _scale_kernel(cntp, h1)
    dis_col = dis_col_p[:N]

    p1 = _agg16(ei, eit, g1)
    g2 = _mid_kernel(p1, g1, dis_col, b1.reshape(1, D_HID), W2)
    p2 = _agg64(ei, eit, g2)
    qv = p2.reshape(2, _QROWS, 128)
    g2p = g2.reshape(_FROWS, 128)
    disp = jnp.broadcast_to(dis_col, (N, D_OUT)).reshape(_FROWS, 128)
    b2p = jnp.concatenate([b2, b2]).reshape(1, 128)
    out = _fin_kernel(qv, g2p, disp, b2p)
    return out.reshape(N, D_OUT)


# speedup vs baseline: 1.0898x; 1.0011x over previous
import functools

import jax
import jax.numpy as jnp
from jax import lax
from jax.experimental import pallas as pl
from jax.experimental.pallas import tpu as pltpu
from jax.experimental.pallas import tpu_sc as plsc

N = 10000
E = 320000
D_IN = 128
D_HID = 16
D_OUT = 64

N_PAD = 10240
CHUNK = 128
CH_FULL = 78
TAIL = 16
EROWS = 2500

_mesh = plsc.VectorSubcoreMesh(core_axis_name="c", subcore_axis_name="s")
_sc_params = pltpu.CompilerParams(use_tc_tiling_on_sc=False)


_GRP = 6
_NG = CH_FULL // _GRP
_W = _GRP * CHUNK


def _stage_indices(ei_hbm, eit_hbm, row, w, idx_v, tail_v, sem):
    pltpu.async_copy(ei_hbm.at[row, pl.ds(w * _NG, _NG)], idx_v, sem)
    pltpu.async_copy(eit_hbm.at[row, pl.ds(w * TAIL, TAIL)], tail_v, sem)


def _wait_indices(ei_hbm, eit_hbm, row, w, idx_v, tail_v, sem):
    pltpu.make_async_copy(ei_hbm.at[row, pl.ds(w * _NG, _NG)], idx_v,
                          sem).wait()
    pltpu.make_async_copy(eit_hbm.at[row, pl.ds(w * TAIL, TAIL)], tail_v,
                          sem).wait()


@functools.partial(
    pl.kernel,
    out_type=jax.ShapeDtypeStruct((2, N_PAD), jnp.float32),
    mesh=_mesh,
    scratch_types=[
        pltpu.VMEM((_NG, _W), jnp.int32),
        pltpu.VMEM((TAIL,), jnp.int32),
        pltpu.VMEM((_W,), jnp.float32),
        pltpu.VMEM((TAIL,), jnp.float32),
        pltpu.VMEM((640,), jnp.float32),
        pltpu.VMEM_SHARED((N_PAD,), jnp.float32),
        pltpu.SemaphoreType.DMA,
        pltpu.SemaphoreType.DMA,
    ],
    compiler_params=_sc_params,
)
def _count_kernel(ei_hbm, eit_hbm, out_hbm, didx_v, dtail_v, ones_v, ones_t,
                  zbuf_v, acc_sh, isem, ssem):
    c = lax.axis_index("c")
    s = lax.axis_index("s")
    w = c * 16 + s
    _stage_indices(ei_hbm, eit_hbm, 1, w, didx_v, dtail_v, isem)

    def fill_body(i, _):
        ones_v[pl.ds(i * 16, 16)] = jnp.ones((16,), jnp.float32)
        return 0
    lax.fori_loop(0, _W // 16, fill_body, 0)
    ones_t[...] = jnp.ones((16,), jnp.float32)

    def zero_body(i, _):
        zbuf_v[pl.ds(i * 16, 16)] = jnp.zeros((16,), jnp.float32)
        return 0
    lax.fori_loop(0, 640 // 16, zero_body, 0)

    _wait_indices(ei_hbm, eit_hbm, 1, w, didx_v, dtail_v, isem)
    pltpu.sync_copy(zbuf_v, acc_sh.at[pl.ds(s * 640, 640)])
    plsc.subcore_barrier()

    def cnt_body(j, _):
        pltpu.make_async_copy(ones_v, acc_sh.at[didx_v.at[j]], ssem).wait()
        pltpu.async_copy(ones_v, acc_sh.at[didx_v.at[j + 2]], ssem, add=True)
        return 0

    for j0 in range(2):
        pltpu.async_copy(ones_v, acc_sh.at[didx_v.at[j0]], ssem, add=True)
    lax.fori_loop(0, _NG - 2, cnt_body, 0)

    def drain_body(j, _):
        pltpu.make_async_copy(ones_v, acc_sh.at[didx_v.at[j]], ssem).wait()
        return 0
    lax.fori_loop(0, 2, drain_body, 0)
    pltpu.sync_copy(ones_t, acc_sh.at[dtail_v], add=True)
    plsc.subcore_barrier()

    pltpu.sync_copy(acc_sh.at[pl.ds(s * 640, 640)],
                    out_hbm.at[c].at[pl.ds(s * 640, 640)])


def _make_agg_kernel(d_feat, subw, nbuf):
    spr = _W // subw
    ng = _NG * spr

    @functools.partial(
        pl.kernel,
        out_type=jax.ShapeDtypeStruct((2, N_PAD, d_feat), jnp.float32),
        mesh=_mesh,
        scratch_types=[
            pltpu.VMEM((_NG, _W), jnp.int32),
            pltpu.VMEM((_NG, _W), jnp.int32),
            pltpu.VMEM((TAIL,), jnp.int32),
            pltpu.VMEM((TAIL,), jnp.int32),
            pltpu.VMEM((nbuf, subw, d_feat), jnp.float32),
            pltpu.VMEM((TAIL, d_feat), jnp.float32),
            pltpu.VMEM((64, d_feat), jnp.float32),
            pltpu.VMEM_SHARED((N_PAD, d_feat), jnp.float32),
            pltpu.SemaphoreType.DMA,
        ] + [pltpu.SemaphoreType.DMA] * (2 * nbuf),
        compiler_params=_sc_params,
    )
    def agg(ei_hbm, eit_hbm, g_hbm, out_hbm, sidx_v, didx_v, stail_v, dtail_v,
            rows_v, trows_v, zbuf_v, acc_sh, isem, *sems):
        gsem = sems[:nbuf]
        ssem = sems[nbuf:]
        c = lax.axis_index("c")
        s = lax.axis_index("s")
        w = c * 16 + s
        _stage_indices(ei_hbm, eit_hbm, 0, w, sidx_v, stail_v, isem)
        _stage_indices(ei_hbm, eit_hbm, 1, w, didx_v, dtail_v, isem)

        def zzero(i, _):
            j = i // (d_feat // 16)
            k = i % (d_feat // 16)
            zbuf_v[j, pl.ds(k * 16, 16)] = jnp.zeros((16,), jnp.float32)
            return 0
        lax.fori_loop(0, 64 * (d_feat // 16), zzero, 0)

        _wait_indices(ei_hbm, eit_hbm, 0, w, sidx_v, stail_v, isem)
        _wait_indices(ei_hbm, eit_hbm, 1, w, didx_v, dtail_v, isem)

        for t in range(10):
            pltpu.async_copy(zbuf_v, acc_sh.at[pl.ds(s * 640 + t * 64, 64)],
                             isem)
        for t in range(10):
            pltpu.make_async_copy(
                zbuf_v, acc_sh.at[pl.ds(s * 640 + t * 64, 64)], isem).wait()
        plsc.subcore_barrier()

        def sl(idx_v, g):
            return idx_v.at[g // spr, pl.ds((g % spr) * subw, subw)]

        def issue(g, b):
            pltpu.async_copy(g_hbm.at[sl(sidx_v, g)], rows_v.at[b], gsem[b])

        def wait_gather(g, b):
            pltpu.make_async_copy(g_hbm.at[sl(sidx_v, g)], rows_v.at[b],
                                  gsem[b]).wait()

        def start_scatter(g, b):
            pltpu.async_copy(rows_v.at[b], acc_sh.at[sl(didx_v, g)], ssem[b],
                             add=True)

        def wait_scatter(g, b):
            pltpu.make_async_copy(rows_v.at[b], acc_sh.at[sl(didx_v, g)],
                                  ssem[b]).wait()

        def step(g, b):
            pb = (b - 1) % nbuf

            @pl.when(g > 0)
            def _():
                wait_scatter(g - 1, pb)

            @pl.when(jnp.logical_and(g > 0, g - 1 + nbuf < ng))
            def _():
                issue(g - 1 + nbuf, pb)

            wait_gather(g, b)
            start_scatter(g, b)

        for b in range(nbuf):
            issue(b, b)

        k_full = ng // nbuf

        def outer_body(o, _):
            for b in range(nbuf):
                step(o * nbuf + b, b)
            return 0
        lax.fori_loop(0, k_full, outer_body, 0)

        for g in range(k_full * nbuf, ng):
            step(g, g % nbuf)
        wait_scatter(ng - 1, (ng - 1) % nbuf)

        pltpu.async_copy(g_hbm.at[stail_v], trows_v, isem)
        pltpu.make_async_copy(g_hbm.at[stail_v], trows_v, isem).wait()
        pltpu.sync_copy(trows_v, acc_sh.at[dtail_v], add=True)
        plsc.subcore_barrier()

        pltpu.sync_copy(acc_sh.at[pl.ds(s * 640, 640)],
                        out_hbm.at[c].at[pl.ds(s * 640, 640)])
    return agg


_agg16 = _make_agg_kernel(D_HID, _W, 2)
_agg64 = _make_agg_kernel(D_OUT, _W // 6, 6)


def _h1_body(x_ref, w1_ref, out_ref):
    out_ref[...] = jnp.dot(x_ref[...], w1_ref[...],
                           preferred_element_type=jnp.float32)


_h1_kernel = pl.pallas_call(
    _h1_body,
    out_shape=jax.ShapeDtypeStruct((N, D_HID), jnp.float32),
)


def _scale_body(cnt_ref, h1_ref, g1_ref, dis_ref):
    deg = jnp.sum(cnt_ref[...], axis=0, keepdims=True) + 1.0
    dis_col = jnp.transpose(lax.rsqrt(deg), (1, 0))
    dis_ref[...] = dis_col
    g1_ref[...] = dis_col[:N] * h1_ref[...]


_scale_kernel = pl.pallas_call(
    _scale_body,
    out_shape=(jax.ShapeDtypeStruct((N, D_HID), jnp.float32),
               jax.ShapeDtypeStruct((N_PAD, 1), jnp.float32)),
)


def _mid_body(p_ref, g1_ref, dis_ref, b1_ref, w2_ref, out_ref):
    agg = p_ref[0, :N, :] + p_ref[1, :N, :]
    a1 = jnp.maximum(dis_ref[...] * (agg + g1_ref[...]) + b1_ref[...], 0.0)
    h2 = jnp.dot(a1, w2_ref[...], preferred_element_type=jnp.float32)
    out_ref[...] = dis_ref[...] * h2


_mid_kernel = pl.pallas_call(
    _mid_body,
    out_shape=jax.ShapeDtypeStruct((N, D_OUT), jnp.float32),
)


_QROWS = N_PAD * D_OUT // 128
_FROWS = N * D_OUT // 128


def _fin_body(q_ref, g2_ref, dis_ref, b2_ref, out_ref):
    agg = q_ref[0, :_FROWS, :] + q_ref[1, :_FROWS, :]
    out_ref[...] = dis_ref[...] * (agg + g2_ref[...]) + b2_ref[...]


_fin_kernel = pl.pallas_call(
    _fin_body,
    out_shape=jax.ShapeDtypeStruct((_FROWS, 128), jnp.float32),
)


def kernel(x, edge_index, W1, b1, W2, b2):
    ei32 = edge_index.astype(jnp.int32)
    ei = ei32[:, :32 * _NG * _W].reshape(2, 32 * _NG, _W)
    eit = ei32[:, 32 * _NG * _W:]

    cntp = _count_kernel(ei, eit)
    h1 = _h1_kernel(x, W1)
    g1, dis_col_p = _scale_kernel(cntp, h1)
    dis_col = dis_col_p[:N]

    p1 = _agg16(ei, eit, g1)
    g2 = _mid_kernel(p1, g1, dis_col, b1.reshape(1, D_HID), W2)
    p2 = _agg64(ei, eit, g2)
    qv = p2.reshape(2, _QROWS, 128)
    g2p = g2.reshape(_FROWS, 128)
    disp = jnp.broadcast_to(dis_col, (N, D_OUT)).reshape(_FROWS, 128)
    b2p = jnp.concatenate([b2, b2]).reshape(1, 128)
    out = _fin_kernel(qv, g2p, disp, b2p)
    return out.reshape(N, D_OUT)
